# baseline (speedup 1.0000x reference)
import jax
import jax.numpy as jnp
from jax.experimental import pallas as pl
from jax.experimental.pallas import tpu as pltpu

BATCH = 16384
CLASSES = 1000
GAMMA = 2.0
BLK = 1024
NB = BATCH // BLK


def _focal_body(x_ref, t_ref, a_ref, out_ref):
    i = pl.program_id(0)
    x = x_ref[...]
    t = t_ref[0, 0, :]
    m = jnp.max(x, axis=0, keepdims=True)
    e = jnp.exp(x - m)
    s = jnp.sum(e, axis=0, keepdims=True)

    row = jax.lax.broadcasted_iota(jnp.int32, (CLASSES, BLK), 0)
    mask = row == t[None, :]
    xt = jnp.sum(jnp.where(mask, x, 0.0), axis=0, keepdims=True)
    a = a_ref[...]
    at = jnp.sum(jnp.where(mask, a, 0.0), axis=0, keepdims=True)

    log_p = (xt - m) - jnp.log(s)
    p = jnp.exp(log_p)
    omp = 1.0 - p
    loss = -at * (omp * omp) * log_p
    part = jnp.sum(loss)

    @pl.when(i == 0)
    def _():
        out_ref[0, 0] = 0.0

    out_ref[0, 0] += part

    @pl.when(i == NB - 1)
    def _():
        out_ref[0, 0] = out_ref[0, 0] * (1.0 / BATCH)


def kernel(inputs, targets, alpha):
    xT = inputs.T
    t3 = targets.reshape(NB, 1, BLK)
    out = pl.pallas_call(
        _focal_body,
        grid=(NB,),
        in_specs=[
            pl.BlockSpec((CLASSES, BLK), lambda i: (0, i)),
            pl.BlockSpec((1, 1, BLK), lambda i: (i, 0, 0)),
            pl.BlockSpec((CLASSES, 1), lambda i: (0, 0)),
        ],
        out_specs=pl.BlockSpec(memory_space=pltpu.SMEM),
        out_shape=jax.ShapeDtypeStruct((1, 1), jnp.float32),
    )(xT, t3, alpha)
    return out[0, 0]


# speedup vs baseline: 1.0579x; 1.0579x over previous
import jax
import jax.numpy as jnp
from jax.experimental import pallas as pl
from jax.experimental.pallas import tpu as pltpu

BATCH = 16384
CLASSES = 1000
GAMMA = 2.0
BLK = 1024
NB = BATCH // BLK


def _focal_body(x_ref, t_ref, a_ref, out_ref):
    i = pl.program_id(0)
    x = x_ref[...]
    t = t_ref[0, 0, :]
    m = jnp.max(x, axis=0, keepdims=True)
    e = jnp.exp(x - m)
    s = jnp.sum(e, axis=0, keepdims=True)

    row = jax.lax.broadcasted_iota(jnp.int32, (CLASSES, BLK), 0)
    onehot = (row == t[None, :]).astype(jnp.float32)
    xt = jnp.sum(x * onehot, axis=0, keepdims=True)
    a = a_ref[...]
    at = jax.lax.dot_general(
        a, onehot, (((0,), (0,)), ((), ())),
        preferred_element_type=jnp.float32,
    )

    log_p = (xt - m) - jnp.log(s)
    p = jnp.exp(log_p)
    omp = 1.0 - p
    loss = -at * (omp * omp) * log_p
    part = jnp.sum(loss)

    @pl.when(i == 0)
    def _():
        out_ref[0, 0] = 0.0

    out_ref[0, 0] += part

    @pl.when(i == NB - 1)
    def _():
        out_ref[0, 0] = out_ref[0, 0] * (1.0 / BATCH)


def kernel(inputs, targets, alpha):
    xT = inputs.T
    t3 = targets.reshape(NB, 1, BLK)
    out = pl.pallas_call(
        _focal_body,
        grid=(NB,),
        in_specs=[
            pl.BlockSpec((CLASSES, BLK), lambda i: (0, i)),
            pl.BlockSpec((1, 1, BLK), lambda i: (i, 0, 0)),
            pl.BlockSpec((CLASSES, 1), lambda i: (0, 0)),
        ],
        out_specs=pl.BlockSpec(memory_space=pltpu.SMEM),
        out_shape=jax.ShapeDtypeStruct((1, 1), jnp.float32),
    )(xT, t3, alpha)
    return out[0, 0]
